# Initial kernel scaffold; baseline (speedup 1.0000x reference)
import jax
import jax.numpy as jnp
from jax.experimental import pallas as pl


def kernel(x, edge_index, W_in, b_in, W_g, b_g, gamma, beta, W_o1, b_o1, W_o2, b_o2, W_o3, b_o3):
    raise NotImplementedError("write your pallas kernel here")


# speedup vs baseline: 12.6816x; 12.6816x over previous
import functools

import jax
import jax.numpy as jnp
from jax import lax
from jax.experimental import pallas as pl
from jax.experimental.pallas import tpu as pltpu
from jax.experimental.pallas import tpu_sc as plsc

N = 50000
E = 800000
H = 64
HH = 32
A = 50048
ROWS_PER_TILE = A // 16
E_PAD = 819200
EV_ROWS = E_PAD // 128
R = 1000
GRID = N // R

_sc_params = pltpu.CompilerParams(use_tc_tiling_on_sc=False)


def _degree_body(dstv_hbm, ones_hbm, zeros_hbm, d0_hbm, d1_hbm, dst_v, ones_v, acc):
    c = lax.axis_index("c")
    s = lax.axis_index("s")
    w = c * 16 + s
    base = s * ROWS_PER_TILE
    pltpu.sync_copy(zeros_hbm.at[pl.ds(base, ROWS_PER_TILE)],
                    acc.at[pl.ds(base, ROWS_PER_TILE)])
    pltpu.sync_copy(ones_hbm, ones_v)
    plsc.subcore_barrier()

    def body(k, _):
        row = w * 200 + k * 8
        pltpu.sync_copy(dstv_hbm.at[pl.ds(row, 8)], dst_v)
        for j in range(8):
            pltpu.sync_copy(ones_v, acc.at[dst_v.at[j]], add=True)
        return _

    lax.fori_loop(0, 25, body, None)
    plsc.subcore_barrier()

    def copy_out(out_ref):
        pltpu.sync_copy(acc.at[pl.ds(base, ROWS_PER_TILE)], out_ref.at[pl.ds(base, ROWS_PER_TILE)])

    @pl.when(c == 0)
    def _():
        copy_out(d0_hbm)

    @pl.when(c == 1)
    def _():
        copy_out(d1_hbm)


def _edge_body(mlo_hbm, mhi_hbm, srcv_hbm, dstv_hbm, slo_hbm, shi_hbm,
               src_v, dst_v, rows_v, acc, sem):
    c = lax.axis_index("c")
    s = lax.axis_index("s")
    base = s * ROWS_PER_TILE

    def run(m_ref, out_ref):
        pltpu.sync_copy(m_ref.at[pl.ds(base, ROWS_PER_TILE)], acc.at[pl.ds(base, ROWS_PER_TILE)])
        plsc.subcore_barrier()

        def body(k, _):
            row = s * 400 + k * 8
            pltpu.sync_copy(srcv_hbm.at[pl.ds(row, 8)], src_v)
            pltpu.sync_copy(dstv_hbm.at[pl.ds(row, 8)], dst_v)
            copies = [
                pltpu.async_copy(m_ref.at[src_v.at[j]], rows_v.at[j], sem[j])
                for j in range(4)
            ]
            for j in range(8):
                copies[j].wait()
                pltpu.sync_copy(rows_v.at[j % 4], acc.at[dst_v.at[j]], add=True)
                if j + 4 < 8:
                    copies.append(pltpu.async_copy(
                        m_ref.at[src_v.at[j + 4]], rows_v.at[j % 4], sem[j % 4]))
            return _

        lax.fori_loop(0, 50, body, None)
        plsc.subcore_barrier()
        pltpu.sync_copy(acc.at[pl.ds(base, ROWS_PER_TILE)], out_ref.at[pl.ds(base, ROWS_PER_TILE)])

    @pl.when(c == 0)
    def _():
        run(mlo_hbm, slo_hbm)

    @pl.when(c == 1)
    def _():
        run(mhi_hbm, shi_hbm)


@functools.lru_cache(maxsize=None)
def _build_sc_kernels():
    mesh = plsc.VectorSubcoreMesh(core_axis_name="c", subcore_axis_name="s",
                                  num_cores=2, num_subcores=16)
    degree_sc = pl.kernel(
        _degree_body,
        out_type=(
            jax.ShapeDtypeStruct((A, HH), jnp.float32),
            jax.ShapeDtypeStruct((A, HH), jnp.float32),
        ),
        mesh=mesh,
        compiler_params=_sc_params,
        scratch_types=[
            pltpu.VMEM((8, 128), jnp.int32),
            pltpu.VMEM((128, HH), jnp.float32),
            pltpu.VMEM_SHARED((A, HH), jnp.float32),
        ],
    )
    edge_sc = pl.kernel(
        _edge_body,
        out_type=(
            jax.ShapeDtypeStruct((A, HH), jnp.float32),
            jax.ShapeDtypeStruct((A, HH), jnp.float32),
        ),
        mesh=mesh,
        compiler_params=_sc_params,
        scratch_types=[
            pltpu.VMEM((8, 128), jnp.int32),
            pltpu.VMEM((8, 128), jnp.int32),
            pltpu.VMEM((4, 128, HH), jnp.float32),
            pltpu.VMEM_SHARED((A, HH), jnp.float32),
            [pltpu.SemaphoreType.DMA] * 4,
        ],
    )
    return degree_sc, edge_sc


def _full(shape):
    return pl.BlockSpec(shape, lambda i: (0,) * len(shape))


def _rows(w):
    return pl.BlockSpec((R, w), lambda i: (i, 0))


def _prep_body(x_ref, d0_ref, d1_ref, win_ref, bin_ref, wg0_ref,
               h_ref, mlo_ref, mhi_ref, dis_ref):
    deg = d0_ref[:, 0:1] + d1_ref[:, 0:1] + 1.0
    dis = lax.rsqrt(deg)
    h = jnp.dot(x_ref[...], win_ref[...], preferred_element_type=jnp.float32) + bin_ref[...]
    m = jnp.dot(h, wg0_ref[...], preferred_element_type=jnp.float32) * dis
    h_ref[...] = h
    dis_ref[...] = dis
    mlo_ref[...] = m[:, :HH]
    mhi_ref[...] = m[:, HH:]


_prep_tc = pl.pallas_call(
    _prep_body,
    grid=(GRID,),
    in_specs=[
        _rows(3),
        _rows(HH),
        _rows(HH),
        _full((3, H)), _full((1, H)), _full((H, H)),
    ],
    out_specs=[
        _rows(H), _rows(HH), _rows(HH),
        pl.BlockSpec((R, 1), lambda i: (i, 0)),
    ],
    out_shape=[
        jax.ShapeDtypeStruct((N, H), jnp.float32),
        jax.ShapeDtypeStruct((A, HH), jnp.float32),
        jax.ShapeDtypeStruct((A, HH), jnp.float32),
        jax.ShapeDtypeStruct((N, 1), jnp.float32),
    ],
)


def _stats_body(h_ref, slo_ref, shi_ref, dis_ref, bg_ref, t_ref, sums_ref):
    sagg = jnp.concatenate([slo_ref[...], shi_ref[...]], axis=1)
    t = h_ref[...] + dis_ref[...] * sagg + bg_ref[...]
    t_ref[...] = t

    @pl.when(pl.program_id(0) == 0)
    def _():
        sums_ref[...] = jnp.zeros_like(sums_ref)

    sums_ref[...] += jnp.concatenate(
        [jnp.sum(t, axis=0)[None], jnp.sum(t * t, axis=0)[None]], axis=0)


_stats_tc = pl.pallas_call(
    _stats_body,
    grid=(GRID,),
    in_specs=[
        _rows(H), _rows(HH), _rows(HH),
        pl.BlockSpec((R, 1), lambda i: (i, 0)),
        _full((1, H)),
    ],
    out_specs=[_rows(H), _full((2, H))],
    out_shape=[
        jax.ShapeDtypeStruct((N, H), jnp.float32),
        jax.ShapeDtypeStruct((2, H), jnp.float32),
    ],
)


def _bn_relu(t, sums, gamma, beta):
    mean = sums[0:1, :] * (1.0 / N)
    var = sums[1:2, :] * (1.0 / N) - mean * mean
    inv = lax.rsqrt(var + 1e-5)
    return jnp.maximum((t - mean) * inv * gamma + beta, 0.0)


def _apply_body(t_ref, sums_ref, g_ref, b_ref, dis_ref, wn_ref,
                h_ref, mlo_ref, mhi_ref):
    hn = _bn_relu(t_ref[...], sums_ref[...], g_ref[...], b_ref[...])
    m = jnp.dot(hn, wn_ref[...], preferred_element_type=jnp.float32) * dis_ref[...]
    h_ref[...] = hn
    mlo_ref[...] = m[:, :HH]
    mhi_ref[...] = m[:, HH:]


_apply_tc = pl.pallas_call(
    _apply_body,
    grid=(GRID,),
    in_specs=[
        _rows(H), _full((2, H)), _full((1, H)), _full((1, H)),
        pl.BlockSpec((R, 1), lambda i: (i, 0)),
        _full((H, H)),
    ],
    out_specs=[_rows(H), _rows(HH), _rows(HH)],
    out_shape=[
        jax.ShapeDtypeStruct((N, H), jnp.float32),
        jax.ShapeDtypeStruct((A, HH), jnp.float32),
        jax.ShapeDtypeStruct((A, HH), jnp.float32),
    ],
)


def _final_body(t_ref, sums_ref, g_ref, b_ref, w1_ref, b1_ref, w2_ref, b2_ref,
                w3_ref, b3_ref, o_ref):
    hn = _bn_relu(t_ref[...], sums_ref[...], g_ref[...], b_ref[...])
    o = jnp.maximum(jnp.dot(hn, w1_ref[...], preferred_element_type=jnp.float32) + b1_ref[...], 0.0)
    o = jnp.maximum(jnp.dot(o, w2_ref[...], preferred_element_type=jnp.float32) + b2_ref[...], 0.0)
    o_ref[...] = jnp.dot(o, w3_ref[...], preferred_element_type=jnp.float32) + b3_ref[...]


_final_tc = pl.pallas_call(
    _final_body,
    grid=(GRID,),
    in_specs=[
        _rows(H), _full((2, H)), _full((1, H)), _full((1, H)),
        _full((H, H)), _full((1, H)),
        _full((H, HH)), _full((1, HH)),
        _full((HH, 8)), _full((1, 8)),
    ],
    out_specs=[_rows(8)],
    out_shape=[jax.ShapeDtypeStruct((N, 8), jnp.float32)],
)


def kernel(x, edge_index, W_in, b_in, W_g, b_g, gamma, beta,
           W_o1, b_o1, W_o2, b_o2, W_o3, b_o3):
    src = edge_index[0]
    dst = edge_index[1]
    pad = E_PAD - E
    srcv = jnp.concatenate([src, jnp.zeros((pad,), jnp.int32)]).reshape(EV_ROWS, 128)
    dstv = jnp.concatenate([dst, jnp.full((pad,), N, jnp.int32)]).reshape(EV_ROWS, 128)

    degree_sc, edge_sc = _build_sc_kernels()
    ones = jnp.ones((128, HH), jnp.float32)
    zerosA = jnp.zeros((A, HH), jnp.float32)
    d0, d1 = degree_sc(dstv, ones, zerosA)

    h, mlo, mhi, dis = _prep_tc(
        x, d0, d1, W_in, b_in.reshape(1, H), W_g[0])

    out = None
    for i in range(4):
        slo, shi = edge_sc(mlo, mhi, srcv, dstv)
        t, sums = _stats_tc(h, slo, shi, dis, b_g[i].reshape(1, H))
        if i < 3:
            h, mlo, mhi = _apply_tc(
                t, sums, gamma[i].reshape(1, H), beta[i].reshape(1, H),
                dis, W_g[i + 1])
        else:
            (out,) = _final_tc(
                t, sums, gamma[i].reshape(1, H), beta[i].reshape(1, H),
                W_o1, b_o1.reshape(1, H), W_o2, b_o2.reshape(1, HH),
                W_o3, b_o3.reshape(1, 8))
    return out
